# baseline (speedup 1.0000x reference)
import functools

import jax
import jax.numpy as jnp
from jax import lax
from jax.experimental import pallas as pl
from jax.experimental.pallas import tpu as pltpu
from jax.experimental.pallas import tpu_sc as plsc

_NUM_FIELDS = 26
_VOCAB = 100000
_EMBED_DIM = 32
_BATCH = 4096

_NUM_CORES = 2
_NUM_SUBCORES = 16
_NUM_WORKERS = _NUM_CORES * _NUM_SUBCORES

_TOTAL_ROWS = _BATCH * _NUM_FIELDS
_ROWS_PER_WORKER = _TOTAL_ROWS // _NUM_WORKERS
_CHUNK = 128
_NUM_CHUNKS = _ROWS_PER_WORKER // _CHUNK
_LANES = 16


def _body(idx_hbm, tab_hbm, out_hbm, gidx_v, soff_v, rows4_v, outc_v, sems):
    wid = lax.axis_index("s") * _NUM_CORES + lax.axis_index("c")
    base = wid * _ROWS_PER_WORKER

    pltpu.sync_copy(idx_hbm.at[pl.ds(base, _ROWS_PER_WORKER)], gidx_v)

    def compute(t, carry):
        q = t * _LANES
        pos = q + lax.iota(jnp.int32, _LANES)
        flat = gidx_v[pl.ds(q, _LANES)] + (pos % _NUM_FIELDS) * _VOCAB
        gidx_v[pl.ds(q, _LANES)] = flat >> 2
        soff_v[pl.ds(q, _LANES)] = (flat & 3) * _EMBED_DIM
        return carry

    lax.fori_loop(0, _ROWS_PER_WORKER // _LANES, compute, 0, unroll=4)

    def fire(j, slot):
        pltpu.make_async_copy(
            tab_hbm.at[gidx_v.at[pl.ds(j * _CHUNK, _CHUNK)]],
            rows4_v.at[slot],
            sems.at[slot],
        ).start()

    fire(0, 0)

    def extract(j, slot):
        rflat = rows4_v.at[slot]
        oflat = outc_v.at[slot]

        def step(t, carry):
            lanes = lax.iota(jnp.int32, _LANES)
            r = t * _LANES + lanes
            soff = soff_v[pl.ds(j * _CHUNK + t * _LANES, _LANES)]

            def cstep(c, carry2):
                cvec = jnp.full((_LANES,), 0, jnp.int32) + c
                vals = plsc.load_gather(rflat, [r, soff + c])
                plsc.store_scatter(oflat, [r, cvec], vals)
                return carry2

            lax.fori_loop(0, _EMBED_DIM, cstep, 0, unroll=8)
            return carry

        lax.fori_loop(0, _CHUNK // _LANES, step, 0)

    def drain_extract_store(j, slot):
        pltpu.make_async_copy(
            tab_hbm.at[gidx_v.at[pl.ds(j * _CHUNK, _CHUNK)]],
            rows4_v.at[slot],
            sems.at[slot],
        ).wait()
        extract(j, slot)
        pltpu.sync_copy(
            outc_v.at[slot],
            out_hbm.at[pl.ds(base + j * _CHUNK, _CHUNK)],
        )

    def pair(h, carry):
        j0 = h * 2
        fire(j0 + 1, 1)
        drain_extract_store(j0, 0)

        @pl.when(j0 + 2 < _NUM_CHUNKS)
        def _():
            fire(j0 + 2, 0)

        drain_extract_store(j0 + 1, 1)
        return carry

    lax.fori_loop(0, _NUM_CHUNKS // 2, pair, 0)


@jax.jit
def _embed(categorical_inputs, tables):
    idx_flat = categorical_inputs.reshape(_TOTAL_ROWS)
    tab4 = tables.reshape(_NUM_FIELDS * _VOCAB * _EMBED_DIM // 128, 128)

    mesh = plsc.VectorSubcoreMesh(core_axis_name="c", subcore_axis_name="s")
    out = pl.kernel(
        _body,
        out_type=jax.ShapeDtypeStruct((_TOTAL_ROWS, _EMBED_DIM), jnp.float32),
        mesh=mesh,
        scratch_types=[
            pltpu.VMEM((_ROWS_PER_WORKER,), jnp.int32),
            pltpu.VMEM((_ROWS_PER_WORKER,), jnp.int32),
            pltpu.VMEM((2, _CHUNK, 128), jnp.float32),
            pltpu.VMEM((2, _CHUNK, _EMBED_DIM), jnp.float32),
            pltpu.SemaphoreType.DMA((2,)),
        ],
        compiler_params=pltpu.CompilerParams(
            use_tc_tiling_on_sc=True, needs_layout_passes=False
        ),
    )(idx_flat, tab4)
    return out.reshape(_BATCH, _NUM_FIELDS * _EMBED_DIM)


def kernel(categorical_inputs, tables):
    return _embed(categorical_inputs, tables)


# speedup vs baseline: 1.0720x; 1.0720x over previous
import functools

import jax
import jax.numpy as jnp
from jax import lax
from jax.experimental import pallas as pl
from jax.experimental.pallas import tpu as pltpu
from jax.experimental.pallas import tpu_sc as plsc

_NUM_FIELDS = 26
_VOCAB = 100000
_EMBED_DIM = 32
_BATCH = 4096

_NUM_CORES = 2
_NUM_SUBCORES = 16
_NUM_WORKERS = _NUM_CORES * _NUM_SUBCORES

_CHUNK = 128
_NUM_BC = _BATCH // _CHUNK
_NUM_TASKS = _NUM_FIELDS * _NUM_BC
_TASKS_PER_WORKER = _NUM_TASKS // _NUM_WORKERS
_LANES = 16


def _body(idx_hbm, tab_hbm, out_hbm, icomp_v, gidx_v, soff_v, rows4_v, outc_v, sems):
    wid = lax.axis_index("s") * _NUM_CORES + lax.axis_index("c")
    t0 = wid * _TASKS_PER_WORKER

    def prep_fire(k, slot):
        t = t0 + k
        f = t // _NUM_BC
        bc = t % _NUM_BC
        pltpu.sync_copy(
            idx_hbm.at[f, pl.ds(bc * _CHUNK, _CHUNK)], icomp_v.at[slot]
        )

        def compute(q, carry):
            s = q * _LANES
            flat = icomp_v[slot, pl.ds(s, _LANES)] + f * _VOCAB
            gidx_v[slot, pl.ds(s, _LANES)] = flat >> 2
            soff_v[slot, pl.ds(s, _LANES)] = (flat & 3) * _EMBED_DIM
            return carry

        lax.fori_loop(0, _CHUNK // _LANES, compute, 0, unroll=4)

        pltpu.make_async_copy(
            tab_hbm.at[gidx_v.at[slot]],
            rows4_v.at[slot],
            sems.at[slot],
        ).start()

    def extract(slot):
        rflat = rows4_v.at[slot]

        def step(q, carry):
            s = q * _LANES
            b = s + lax.iota(jnp.int32, _LANES)
            soff = soff_v[slot, pl.ds(s, _LANES)]

            def estep(e, carry2):
                vals = plsc.load_gather(rflat, [b, soff + e])
                outc_v[slot, e, pl.ds(s, _LANES)] = vals
                return carry2

            lax.fori_loop(0, _EMBED_DIM, estep, 0, unroll=8)
            return carry

        lax.fori_loop(0, _CHUNK // _LANES, step, 0)

    def drain_extract_store(k, slot):
        t = t0 + k
        f = t // _NUM_BC
        bc = t % _NUM_BC
        pltpu.make_async_copy(
            tab_hbm.at[gidx_v.at[slot]],
            rows4_v.at[slot],
            sems.at[slot],
        ).wait()
        extract(slot)
        pltpu.sync_copy(
            outc_v.at[slot],
            out_hbm.at[f, :, pl.ds(bc * _CHUNK, _CHUNK)],
        )

    prep_fire(0, 0)

    def pair(h, carry):
        k0 = h * 2
        prep_fire(k0 + 1, 1)
        drain_extract_store(k0, 0)

        @pl.when(k0 + 2 < _TASKS_PER_WORKER)
        def _():
            prep_fire(k0 + 2, 0)

        drain_extract_store(k0 + 1, 1)
        return carry

    lax.fori_loop(0, _TASKS_PER_WORKER // 2, pair, 0)


@jax.jit
def _embed(categorical_inputs, tables):
    idx_t = categorical_inputs.T
    tab4 = tables.reshape(_NUM_FIELDS * _VOCAB * _EMBED_DIM // 128, 128)

    mesh = plsc.VectorSubcoreMesh(core_axis_name="c", subcore_axis_name="s")
    out = pl.kernel(
        _body,
        out_type=jax.ShapeDtypeStruct(
            (_NUM_FIELDS, _EMBED_DIM, _BATCH), jnp.float32
        ),
        mesh=mesh,
        scratch_types=[
            pltpu.VMEM((2, _CHUNK), jnp.int32),
            pltpu.VMEM((2, _CHUNK), jnp.int32),
            pltpu.VMEM((2, _CHUNK), jnp.int32),
            pltpu.VMEM((2, _CHUNK, 128), jnp.float32),
            pltpu.VMEM((2, _EMBED_DIM, _CHUNK), jnp.float32),
            pltpu.SemaphoreType.DMA((2,)),
        ],
        compiler_params=pltpu.CompilerParams(
            use_tc_tiling_on_sc=True, needs_layout_passes=False
        ),
    )(idx_t, tab4)
    return out.reshape(_NUM_FIELDS * _EMBED_DIM, _BATCH).T


def kernel(categorical_inputs, tables):
    return _embed(categorical_inputs, tables)


# speedup vs baseline: 1.1279x; 1.0522x over previous
import functools

import jax
import jax.numpy as jnp
from jax import lax
from jax.experimental import pallas as pl
from jax.experimental.pallas import tpu as pltpu
from jax.experimental.pallas import tpu_sc as plsc

_NUM_FIELDS = 26
_VOCAB = 100000
_EMBED_DIM = 32
_BATCH = 4096

_NUM_CORES = 2
_NUM_SUBCORES = 16
_NUM_WORKERS = _NUM_CORES * _NUM_SUBCORES

_TOTAL_ROWS = _BATCH * _NUM_FIELDS
_ROWS_PER_WORKER = _TOTAL_ROWS // _NUM_WORKERS
_CHUNK = 128
_NUM_CHUNKS = _ROWS_PER_WORKER // _CHUNK
_LANES = 16


def _body(idx_hbm, tab_hbm, out_hbm, idx_raw_v, idx_flat_v, rows_v, sem):
    wid = lax.axis_index("s") * _NUM_CORES + lax.axis_index("c")
    base = wid * _ROWS_PER_WORKER

    pltpu.sync_copy(idx_hbm.at[pl.ds(base, _ROWS_PER_WORKER)], idx_raw_v)

    def compute(t, carry):
        q = t * _LANES
        pos = q + lax.iota(jnp.int32, _LANES)
        off = (pos % _NUM_FIELDS) * _VOCAB
        idx_flat_v[pl.ds(q, _LANES)] = idx_raw_v[pl.ds(q, _LANES)] + off
        return carry

    lax.fori_loop(0, _ROWS_PER_WORKER // _LANES, compute, 0, unroll=4)

    def fire(j, carry):
        q = j * _CHUNK
        pltpu.make_async_copy(
            tab_hbm.at[idx_flat_v.at[pl.ds(q, _CHUNK)]],
            rows_v.at[pl.ds(q, _CHUNK)],
            sem,
        ).start()
        return carry

    lax.fori_loop(0, _NUM_CHUNKS, fire, 0)

    pltpu.make_async_copy(
        out_hbm.at[pl.ds(base, _ROWS_PER_WORKER)], rows_v, sem
    ).wait()

    pltpu.sync_copy(rows_v, out_hbm.at[pl.ds(base, _ROWS_PER_WORKER)])


@jax.jit
def _embed(categorical_inputs, tables):
    idx_flat = categorical_inputs.reshape(_TOTAL_ROWS)
    tab_flat = tables.reshape(_NUM_FIELDS * _VOCAB, _EMBED_DIM)

    mesh = plsc.VectorSubcoreMesh(core_axis_name="c", subcore_axis_name="s")
    out = pl.kernel(
        _body,
        out_type=jax.ShapeDtypeStruct((_TOTAL_ROWS, _EMBED_DIM), jnp.float32),
        mesh=mesh,
        scratch_types=[
            pltpu.VMEM((_ROWS_PER_WORKER,), jnp.int32),
            pltpu.VMEM((_ROWS_PER_WORKER,), jnp.int32),
            pltpu.VMEM((_ROWS_PER_WORKER, _EMBED_DIM), jnp.float32),
            pltpu.SemaphoreType.DMA,
        ],
        compiler_params=pltpu.CompilerParams(use_tc_tiling_on_sc=False),
    )(idx_flat, tab_flat)
    return out.reshape(_BATCH, _NUM_FIELDS * _EMBED_DIM)


def kernel(categorical_inputs, tables):
    return _embed(categorical_inputs, tables)
